# baseline (speedup 1.0000x reference)
import jax
import jax.numpy as jnp
from jax.experimental import pallas as pl
from jax.experimental.pallas import tpu as pltpu

_BLOCK_ROWS = 512


def _router_body(x_ref, w1_ref, b1_ref, w2_ref, b2_ref, o_ref):
    xb = x_ref[...].astype(jnp.bfloat16)
    h = jnp.dot(xb, w1_ref[...], preferred_element_type=jnp.float32)
    h = jnp.maximum(h + b1_ref[...], 0.0)
    logits = jnp.sum(h * w2_ref[...], axis=1, keepdims=True) + b2_ref[...]
    o_ref[...] = jax.nn.sigmoid(logits)


def kernel(x, W1, b1, W2, b2):
    n_tokens, input_dim = x.shape
    hidden_dim = W1.shape[1]
    block = _BLOCK_ROWS
    grid = n_tokens // block

    w1b = W1.astype(jnp.bfloat16)
    b1r = b1.reshape(1, hidden_dim)
    w2r = W2.reshape(1, hidden_dim)
    b2r = b2.reshape(1, 1)

    return pl.pallas_call(
        _router_body,
        grid=(grid,),
        in_specs=[
            pl.BlockSpec((block, input_dim), lambda i: (i, 0)),
            pl.BlockSpec((input_dim, hidden_dim), lambda i: (0, 0)),
            pl.BlockSpec((1, hidden_dim), lambda i: (0, 0)),
            pl.BlockSpec((1, hidden_dim), lambda i: (0, 0)),
            pl.BlockSpec((1, 1), lambda i: (0, 0)),
        ],
        out_specs=pl.BlockSpec((block, 1), lambda i: (i, 0)),
        out_shape=jax.ShapeDtypeStruct((n_tokens, 1), jnp.float32),
        compiler_params=pltpu.CompilerParams(
            dimension_semantics=("parallel",),
        ),
    )(x, w1b, b1r, w2r, b2r)


# speedup vs baseline: 1.0054x; 1.0054x over previous
import jax
import jax.numpy as jnp
from jax.experimental import pallas as pl
from jax.experimental.pallas import tpu as pltpu

_BLOCK_ROWS = 512


def _router_body(x_ref, w1_ref, b1_ref, w2_ref, b2_ref, o_ref):
    h = jnp.dot(x_ref[...], w1_ref[...], preferred_element_type=jnp.float32)
    h = jnp.maximum(h + b1_ref[...], 0.0)
    logits = jnp.sum(h * w2_ref[...], axis=1, keepdims=True) + b2_ref[...]
    o_ref[...] = jax.nn.sigmoid(logits)


def kernel(x, W1, b1, W2, b2):
    n_tokens, input_dim = x.shape
    hidden_dim = W1.shape[1]
    block = _BLOCK_ROWS
    grid = n_tokens // block

    w1b = W1.astype(jnp.bfloat16)
    b1r = b1.reshape(1, hidden_dim)
    w2r = W2.reshape(1, hidden_dim)
    b2r = b2.reshape(1, 1)

    return pl.pallas_call(
        _router_body,
        grid=(grid,),
        in_specs=[
            pl.BlockSpec((block, input_dim), lambda i: (i, 0)),
            pl.BlockSpec((input_dim, hidden_dim), lambda i: (0, 0)),
            pl.BlockSpec((1, hidden_dim), lambda i: (0, 0)),
            pl.BlockSpec((1, hidden_dim), lambda i: (0, 0)),
            pl.BlockSpec((1, 1), lambda i: (0, 0)),
        ],
        out_specs=pl.BlockSpec((block, 1), lambda i: (i, 0)),
        out_shape=jax.ShapeDtypeStruct((n_tokens, 1), jnp.float32),
        compiler_params=pltpu.CompilerParams(
            dimension_semantics=("parallel",),
        ),
    )(x, w1b, b1r, w2r, b2r)


# speedup vs baseline: 1.0133x; 1.0079x over previous
import jax
import jax.numpy as jnp
from jax.experimental import pallas as pl
from jax.experimental.pallas import tpu as pltpu

_BLOCK_ROWS = 1024


def _router_body(x_ref, w1_ref, b1_ref, w2_ref, b2_ref, o_ref):
    h = jnp.dot(x_ref[...], w1_ref[...], preferred_element_type=jnp.float32)
    h = jnp.maximum(h + b1_ref[...], 0.0)
    logits = jnp.sum(h * w2_ref[...], axis=1, keepdims=True) + b2_ref[...]
    o_ref[...] = jax.nn.sigmoid(logits)


def kernel(x, W1, b1, W2, b2):
    n_tokens, input_dim = x.shape
    hidden_dim = W1.shape[1]
    block = _BLOCK_ROWS
    grid = n_tokens // block

    w1b = W1.astype(jnp.bfloat16)
    b1r = b1.reshape(1, hidden_dim)
    w2r = W2.reshape(1, hidden_dim)
    b2r = b2.reshape(1, 1)

    return pl.pallas_call(
        _router_body,
        grid=(grid,),
        in_specs=[
            pl.BlockSpec((block, input_dim), lambda i: (i, 0)),
            pl.BlockSpec((input_dim, hidden_dim), lambda i: (0, 0)),
            pl.BlockSpec((1, hidden_dim), lambda i: (0, 0)),
            pl.BlockSpec((1, hidden_dim), lambda i: (0, 0)),
            pl.BlockSpec((1, 1), lambda i: (0, 0)),
        ],
        out_specs=pl.BlockSpec((block, 1), lambda i: (i, 0)),
        out_shape=jax.ShapeDtypeStruct((n_tokens, 1), jnp.float32),
        compiler_params=pltpu.CompilerParams(
            dimension_semantics=("parallel",),
        ),
    )(x, w1b, b1r, w2r, b2r)
